# Initial kernel scaffold; baseline (speedup 1.0000x reference)
import jax
import jax.numpy as jnp
from jax.experimental import pallas as pl


def kernel(x, edge_idx, We1, be1, We2, be2, Wn1, bn1, Wn2, bn2, Wn3, bn3):
    raise NotImplementedError("write your pallas kernel here")


# speedup vs baseline: 2.5855x; 2.5855x over previous
import functools

import jax
import jax.numpy as jnp
from jax import lax
from jax.experimental import pallas as pl
from jax.experimental.pallas import tpu as pltpu
from jax.experimental.pallas import tpu_sc as plsc

N = 10000
D = 128
E = 320000
NC, NS = 2, 16
NW = NC * NS
CH = 128

E_PAD = 327680
EPW = E_PAD // NW
G_ROWS = 2 * E_PAD
GPW = G_ROWS // NW
GCH = GPW // CH
SCH = EPW // CH
N_PAD = 10240
NPS = N_PAD // NS

def _mesh():
    return plsc.VectorSubcoreMesh(
        core_axis_name="c", subcore_axis_name="s",
        num_cores=NC, num_subcores=NS)


def _softplus(v):
    return jnp.logaddexp(v, 0.0)


def _tables_body(x_ref, w_ref, b_ref, out_ref):
    out_ref[0] = jnp.dot(x_ref[...], w_ref[0],
                         preferred_element_type=jnp.float32) + b_ref[0]


def _build_tables(xf, w_stack, b_stack):
    blk = 1000
    return pl.pallas_call(
        _tables_body,
        grid=(2, N // blk),
        in_specs=[
            pl.BlockSpec((blk, D), lambda j, i: (i, 0)),
            pl.BlockSpec((1, D, D), lambda j, i: (j, 0, 0)),
            pl.BlockSpec((1, 1, D), lambda j, i: (j, 0, 0)),
        ],
        out_specs=pl.BlockSpec((1, blk, D), lambda j, i: (j, i, 0)),
        out_shape=jax.ShapeDtypeStruct((2, N, D), jnp.float32),
    )(xf, w_stack, b_stack)


def _gather_body(tbl_hbm, gidx_hbm, out_hbm, idx_v, buf0, buf1, sem0, sem1):
    c = lax.axis_index("c")
    s = lax.axis_index("s")
    wid = s * NC + c
    base = wid * GPW
    pltpu.sync_copy(gidx_hbm.at[wid], idx_v)
    bufs = (buf0, buf1)
    sems = (sem0, sem1)
    pltpu.async_copy(tbl_hbm.at[idx_v.at[0]], bufs[0], sems[0])

    @pl.loop(0, GCH, step=2)
    def _(j):
        for t in range(2):
            jj = j + t

            @pl.when(jj + 1 < GCH)
            def _():
                pltpu.async_copy(tbl_hbm.at[idx_v.at[jj + 1]],
                                 bufs[1 - t], sems[1 - t])

            pltpu.make_async_copy(tbl_hbm.at[idx_v.at[jj]],
                                  bufs[t], sems[t]).wait()
            pltpu.sync_copy(bufs[t], out_hbm.at[pl.ds(base + jj * CH, CH)])


@functools.cache
def _gather_k():
    return pl.kernel(
        _gather_body,
        out_type=jax.ShapeDtypeStruct((G_ROWS, D), jnp.float32),
        mesh=_mesh(),
        scratch_types=[
            pltpu.VMEM((GCH, CH), jnp.int32),
            pltpu.VMEM((CH, D), jnp.float32),
            pltpu.VMEM((CH, D), jnp.float32),
            pltpu.SemaphoreType.DMA,
            pltpu.SemaphoreType.DMA,
        ],
    )


def _edge_body(a_ref, b_ref, w_ref, bias_ref, out_ref):
    h = _softplus(a_ref[...] + b_ref[...])
    m = jnp.dot(h, w_ref[...], preferred_element_type=jnp.float32)
    out_ref[...] = _softplus(m + bias_ref[...])


def _edge_mlp(gout, We2, be2r):
    blk = 2048
    nblk = E_PAD // blk
    return pl.pallas_call(
        _edge_body,
        grid=(nblk,),
        in_specs=[
            pl.BlockSpec((blk, D), lambda j: (j, 0)),
            pl.BlockSpec((blk, D), lambda j: (j + nblk, 0)),
            pl.BlockSpec((D, D), lambda j: (0, 0)),
            pl.BlockSpec((1, D), lambda j: (0, 0)),
        ],
        out_specs=pl.BlockSpec((blk, D), lambda j: (j, 0)),
        out_shape=jax.ShapeDtypeStruct((E_PAD, D), jnp.float32),
    )(gout, gout, We2, be2r)


def _scatter_body(m2_hbm, ridx_hbm, zeros_hbm, out_hbm,
                  idx_v, buf0, buf1, acc, sem0, sem1):
    c = lax.axis_index("c")
    s = lax.axis_index("s")
    wid = s * NC + c
    base = wid * EPW
    pltpu.sync_copy(zeros_hbm.at[pl.ds(s * NPS, NPS)],
                    acc.at[pl.ds(s * NPS, NPS)])
    pltpu.sync_copy(ridx_hbm.at[wid], idx_v)
    plsc.subcore_barrier()
    bufs = (buf0, buf1)
    sems = (sem0, sem1)
    pltpu.async_copy(m2_hbm.at[pl.ds(base, CH)], bufs[0], sems[0])

    @pl.loop(0, SCH, step=2)
    def _(j):
        for t in range(2):
            jj = j + t

            @pl.when(jj + 1 < SCH)
            def _():
                pltpu.async_copy(m2_hbm.at[pl.ds(base + (jj + 1) * CH, CH)],
                                 bufs[1 - t], sems[1 - t])

            pltpu.make_async_copy(m2_hbm.at[pl.ds(base + jj * CH, CH)],
                                  bufs[t], sems[t]).wait()
            pltpu.sync_copy(bufs[t], acc.at[idx_v.at[jj]], add=True)

    plsc.subcore_barrier()
    pltpu.sync_copy(acc.at[pl.ds(s * NPS, NPS)],
                    out_hbm.at[c, pl.ds(s * NPS, NPS)])


@functools.cache
def _scatter_k():
    return pl.kernel(
        _scatter_body,
        out_type=jax.ShapeDtypeStruct((NC, N_PAD, D), jnp.float32),
        mesh=_mesh(),
        scratch_types=[
            pltpu.VMEM((SCH, CH), jnp.int32),
            pltpu.VMEM((CH, D), jnp.float32),
            pltpu.VMEM((CH, D), jnp.float32),
            pltpu.VMEM_SHARED((N_PAD, D), jnp.float32),
            pltpu.SemaphoreType.DMA,
            pltpu.SemaphoreType.DMA,
        ],
    )


def _node_body(x_ref, p0_ref, p1_ref, w1x_ref, w1a_ref, b1_ref,
               w2_ref, b2_ref, w3_ref, b3_ref, out_ref):
    agg = p0_ref[0] + p1_ref[0]
    h = _softplus(jnp.dot(x_ref[...], w1x_ref[...],
                          preferred_element_type=jnp.float32)
                  + jnp.dot(agg, w1a_ref[...],
                            preferred_element_type=jnp.float32)
                  + b1_ref[...])
    h = _softplus(jnp.dot(h, w2_ref[...],
                          preferred_element_type=jnp.float32) + b2_ref[...])
    out_ref[...] = jnp.dot(h, w3_ref[...],
                           preferred_element_type=jnp.float32) + b3_ref[...]


def _node_mlp(xf, partials, Wn1x, Wn1a, bn1r, Wn2, bn2r, Wn3, bn3r):
    blk = 1000
    full = lambda j: (0, 0)
    return pl.pallas_call(
        _node_body,
        grid=(N // blk,),
        in_specs=[
            pl.BlockSpec((blk, D), lambda j: (j, 0)),
            pl.BlockSpec((1, blk, D), lambda j: (0, j, 0)),
            pl.BlockSpec((1, blk, D), lambda j: (1, j, 0)),
            pl.BlockSpec((D, D), full),
            pl.BlockSpec((D, D), full),
            pl.BlockSpec((1, D), full),
            pl.BlockSpec((D, D), full),
            pl.BlockSpec((1, D), full),
            pl.BlockSpec((D, D), full),
            pl.BlockSpec((1, D), full),
        ],
        out_specs=pl.BlockSpec((blk, D), lambda j: (j, 0)),
        out_shape=jax.ShapeDtypeStruct((N, D), jnp.float32),
    )(xf, partials, partials, Wn1x, Wn1a, bn1r, Wn2, bn2r, Wn3, bn3r)


def kernel(x, edge_idx, We1, be1, We2, be2, Wn1, bn1, Wn2, bn2, Wn3, bn3):
    batch = x.shape[0]
    xf = x.reshape(batch * N, D)

    row = edge_idx[0].astype(jnp.int32)
    col = edge_idx[1].astype(jnp.int32)

    arow = jnp.zeros((E_PAD,), jnp.int32).at[:E].set(row)
    bcol = jnp.full((E_PAD,), N, jnp.int32).at[:E].set(col + N)
    g_idx = jnp.concatenate([arow, bcol]).reshape(NW, GCH, CH)
    row_sc = jnp.full((E_PAD,), N, jnp.int32).at[:E].set(row)
    row_sc = row_sc.reshape(NW, SCH, CH)

    w_stack = jnp.stack([We1[:D], We1[D:]])
    b_stack = jnp.stack([be1.reshape(1, D),
                         jnp.zeros((1, D), jnp.float32)])

    tables = _build_tables(xf, w_stack, b_stack)
    gout = _gather_k()(tables.reshape(2 * N, D), g_idx)
    m2 = _edge_mlp(gout, We2, be2.reshape(1, D))
    zeros = jnp.zeros((N_PAD, D), jnp.float32)
    partials = _scatter_k()(m2, row_sc, zeros)
    h = _node_mlp(xf, partials, Wn1[:D], Wn1[D:], bn1.reshape(1, D),
                  Wn2, bn2.reshape(1, D), Wn3, bn3.reshape(1, D))
    return h.reshape(batch, N, D)
